# baseline (speedup 1.0000x reference)
import functools

import jax
import jax.numpy as jnp
from jax import lax
from jax.experimental import pallas as pl
from jax.experimental.pallas import tpu as pltpu
from jax.experimental.pallas import tpu_sc as plsc

N = 10000
F = 128
E = 320000
ROW = 2 * F
NC = 2
NS = 16
NW = NC * NS
EPW = E // NW
C = 80
GPC = C // 16
NCHUNK = EPW // C
NP = 10240
NT = NP + 128
RPT = NT // NS

_BLK = 1000
_GRID = N // _BLK


def _prep_body(x_ref, w0_ref, w1_ref, root_ref, b_ref, t_ref, r_ref):
    xb = x_ref[...]
    t_ref[:, :F] = jnp.dot(xb, w0_ref[...], preferred_element_type=jnp.float32)
    t_ref[:, F:] = jnp.dot(xb, w1_ref[...], preferred_element_type=jnp.float32)
    r_ref[...] = (
        jnp.dot(xb, root_ref[...], preferred_element_type=jnp.float32)
        + b_ref[...]
    )


def _tc_prep(x, w0, w1, root, b):
    return pl.pallas_call(
        _prep_body,
        grid=(_GRID,),
        in_specs=[
            pl.BlockSpec((_BLK, F), lambda i: (i, 0)),
            pl.BlockSpec((F, F), lambda i: (0, 0)),
            pl.BlockSpec((F, F), lambda i: (0, 0)),
            pl.BlockSpec((F, F), lambda i: (0, 0)),
            pl.BlockSpec((1, F), lambda i: (0, 0)),
        ],
        out_specs=[
            pl.BlockSpec((_BLK, ROW), lambda i: (i, 0)),
            pl.BlockSpec((_BLK, F), lambda i: (i, 0)),
        ],
        out_shape=[
            jax.ShapeDtypeStruct((N, ROW), jnp.float32),
            jax.ShapeDtypeStruct((N, F), jnp.float32),
        ],
    )(x, w0, w1, root, b)


def _mid_body(p_ref, d_ref, r_ref, w0_ref, w1_ref, root_ref, b_ref,
              t_ref, r2_ref):
    s = p_ref[0] + p_ref[1]
    deg = jnp.maximum(jnp.sum(d_ref[...], axis=1), 1.0)[:, None]
    h = s / deg + r_ref[...]
    t_ref[:, :F] = jnp.dot(h, w0_ref[...], preferred_element_type=jnp.float32)
    t_ref[:, F:] = jnp.dot(h, w1_ref[...], preferred_element_type=jnp.float32)
    r2_ref[...] = (
        jnp.dot(h, root_ref[...], preferred_element_type=jnp.float32)
        + b_ref[...]
    )


def _tc_mid(parts, deg, r, w0, w1, root, b):
    return pl.pallas_call(
        _mid_body,
        grid=(_GRID,),
        in_specs=[
            pl.BlockSpec((NC, _BLK, F), lambda i: (0, i, 0)),
            pl.BlockSpec((_BLK, NC), lambda i: (i, 0)),
            pl.BlockSpec((_BLK, F), lambda i: (i, 0)),
            pl.BlockSpec((F, F), lambda i: (0, 0)),
            pl.BlockSpec((F, F), lambda i: (0, 0)),
            pl.BlockSpec((F, F), lambda i: (0, 0)),
            pl.BlockSpec((1, F), lambda i: (0, 0)),
        ],
        out_specs=[
            pl.BlockSpec((_BLK, ROW), lambda i: (i, 0)),
            pl.BlockSpec((_BLK, F), lambda i: (i, 0)),
        ],
        out_shape=[
            jax.ShapeDtypeStruct((N, ROW), jnp.float32),
            jax.ShapeDtypeStruct((N, F), jnp.float32),
        ],
    )(parts, deg, r, w0, w1, root, b)


def _fin_body(p_ref, d_ref, r_ref, o_ref):
    s = p_ref[0] + p_ref[1]
    deg = jnp.maximum(jnp.sum(d_ref[...], axis=1), 1.0)[:, None]
    o_ref[...] = s / deg + r_ref[...]


def _tc_fin(parts, deg, r):
    return pl.pallas_call(
        _fin_body,
        grid=(_GRID,),
        in_specs=[
            pl.BlockSpec((NC, _BLK, F), lambda i: (0, i, 0)),
            pl.BlockSpec((_BLK, NC), lambda i: (i, 0)),
            pl.BlockSpec((_BLK, F), lambda i: (i, 0)),
        ],
        out_specs=pl.BlockSpec((_BLK, F), lambda i: (i, 0)),
        out_shape=jax.ShapeDtypeStruct((N, F), jnp.float32),
    )(parts, deg, r)


def _sc_body(with_deg, t_hbm, edata_hbm, u_hbm, zeros_hbm, out_hbm,
             eb0_v, eb1_v, ub0_v, ub1_v, gat_v, msg_v, moh_v, dch_v, didx_v,
             acc_sh, gsem, isem0, isem1, msem, hsem):
    cid = lax.axis_index("c")
    sid = lax.axis_index("s")
    wid = sid * NC + cid
    cbase = wid * NCHUNK
    last = NCHUNK - 1

    ebufs = (eb0_v, eb1_v)
    ubufs = (ub0_v, ub1_v)
    isems = (isem0, isem1)

    def issue_idx(k, b):
        pltpu.async_copy(
            edata_hbm.at[pl.ds((cbase + k) * (2 * C), 2 * C)],
            ebufs[b], isems[b])
        pltpu.async_copy(
            u_hbm.at[pl.ds((cbase + k) * C, C)], ubufs[b], isems[b])

    def wait_idx(b):
        pltpu.make_async_copy(
            edata_hbm.at[pl.ds(cbase * (2 * C), 2 * C)],
            ebufs[b], isems[b]).wait()
        pltpu.make_async_copy(
            u_hbm.at[pl.ds(cbase * C, C)], ubufs[b], isems[b]).wait()

    def issue_gat(b):
        pltpu.async_copy(
            t_hbm.at[ebufs[b].at[pl.ds(0, C)]], gat_v, gsem)

    def wait_gat(b):
        pltpu.make_async_copy(
            t_hbm.at[ebufs[b].at[pl.ds(0, C)]], gat_v, gsem).wait()

    issue_idx(0, 0)
    issue_idx(1, 1)
    wait_idx(0)
    issue_gat(0)
    pltpu.sync_copy(zeros_hbm.at[pl.ds(sid * RPT, RPT)],
                    acc_sh.at[pl.ds(sid * RPT, RPT)])
    plsc.subcore_barrier()

    pltpu.sync_copy(zeros_hbm.at[pl.ds(0, C)], msg_v)

    def seed_idx(g, c2):
        dch_v[pl.ds(g * 16, 16)] = lax.iota(jnp.int32, 16) + g * 16
        return c2

    lax.fori_loop(0, GPC, seed_idx, 0)
    pltpu.async_copy(msg_v, acc_sh.at[dch_v], msem, add=True)
    if with_deg:
        pltpu.sync_copy(zeros_hbm.at[pl.ds(0, C)], moh_v)
        lax.fori_loop(0, GPC, lambda g, c2: (
            didx_v.__setitem__(pl.ds(g * 16, 16),
                               lax.iota(jnp.int32, 16) + g * 16) or c2), 0)
        pltpu.async_copy(moh_v, acc_sh.at[didx_v], hsem, add=True)

    def wait_scatters():
        pltpu.make_async_copy(msg_v, acc_sh.at[dch_v], msem).wait()
        if with_deg:
            pltpu.make_async_copy(moh_v, acc_sh.at[didx_v], hsem).wait()

    def do_chunk(c, b):
        eb_v = ebufs[b]
        ub_v = ubufs[b]
        wait_gat(b)
        wait_scatters()

        def g_body(g, c2):
            dst16 = eb_v[pl.ds(C + g * 16, 16)]
            dch_v[pl.ds(g * 16, 16)] = dst16
            if with_deg:
                didx_v[pl.ds(g * 16, 16)] = jnp.right_shift(dst16, 7) + NP
                col16 = jnp.bitwise_and(dst16, 127)
            u16 = ub_v[pl.ds(g * 16, 16)]
            u16 = jnp.minimum(jnp.maximum(u16, 0.0), 1.0)
            for e2 in range(16):
                e = g * 16 + e2
                ub = jnp.full((16,), u16[e2], dtype=jnp.float32)
                for j in range(F // 16):
                    a = gat_v[e, pl.ds(j * 16, 16)]
                    bb = gat_v[e, pl.ds(F + j * 16, 16)]
                    msg_v[e, pl.ds(j * 16, 16)] = a + ub * (bb - a)
                if with_deg:
                    cb = jnp.full((16,), col16[e2], dtype=jnp.int32)
                    for j in range(F // 16):
                        lane = lax.iota(jnp.int32, 16) + (16 * j)
                        moh_v[e, pl.ds(j * 16, 16)] = jnp.where(
                            lane == cb, 1.0, 0.0)
            return c2

        lax.fori_loop(0, GPC, g_body, 0)
        wait_idx(1 - b)
        issue_gat(1 - b)
        pltpu.async_copy(msg_v, acc_sh.at[dch_v], msem, add=True)
        if with_deg:
            pltpu.async_copy(moh_v, acc_sh.at[didx_v], hsem, add=True)
        issue_idx(jnp.minimum(c + 2, last), b)

    def pair_body(i, carry):
        do_chunk(2 * i, 0)
        do_chunk(2 * i + 1, 1)
        return carry

    lax.fori_loop(0, NCHUNK // 2, pair_body, 0)
    do_chunk(last, 0)
    wait_idx(0)
    wait_gat(1)
    wait_scatters()
    plsc.subcore_barrier()

    pltpu.sync_copy(acc_sh.at[pl.ds(sid * RPT, RPT)],
                    out_hbm.at[cid, pl.ds(sid * RPT, RPT)])


_SC_MESH = plsc.VectorSubcoreMesh(core_axis_name="c", subcore_axis_name="s")


def _mk_scratch(with_deg):
    mrows = C if with_deg else 16
    return [
        pltpu.VMEM((2 * C,), jnp.int32),
        pltpu.VMEM((2 * C,), jnp.int32),
        pltpu.VMEM((C,), jnp.float32),
        pltpu.VMEM((C,), jnp.float32),
        pltpu.VMEM((C, ROW), jnp.float32),
        pltpu.VMEM((C, F), jnp.float32),
        pltpu.VMEM((mrows, F), jnp.float32),
        pltpu.VMEM((C,), jnp.int32),
        pltpu.VMEM((C,), jnp.int32),
        pltpu.VMEM_SHARED((NT, F), jnp.float32),
        pltpu.SemaphoreType.DMA,
        pltpu.SemaphoreType.DMA,
        pltpu.SemaphoreType.DMA,
        pltpu.SemaphoreType.DMA,
        pltpu.SemaphoreType.DMA,
    ]


@functools.partial(
    pl.kernel,
    mesh=_SC_MESH,
    out_type=jax.ShapeDtypeStruct((NC, NT, F), jnp.float32),
    scratch_types=_mk_scratch(True),
)
def _sc_agg_deg(t_hbm, edata_hbm, u_hbm, zeros_hbm, out_hbm, *rest):
    _sc_body(True, t_hbm, edata_hbm, u_hbm, zeros_hbm, out_hbm, *rest)


@functools.partial(
    pl.kernel,
    mesh=_SC_MESH,
    out_type=jax.ShapeDtypeStruct((NC, NT, F), jnp.float32),
    scratch_types=_mk_scratch(False),
)
def _sc_agg(t_hbm, edata_hbm, u_hbm, zeros_hbm, out_hbm, *rest):
    _sc_body(False, t_hbm, edata_hbm, u_hbm, zeros_hbm, out_hbm, *rest)


def kernel(t, x, edge_index, edge_attr, W1, root1, b1, W2, root2, b2):
    src = edge_index[0].astype(jnp.int32)
    dst = edge_index[1].astype(jnp.int32)
    u = edge_attr[:, 0].astype(jnp.float32)
    nch = E // C
    edata = jnp.stack(
        [src.reshape(nch, C), dst.reshape(nch, C)], axis=1).reshape(-1)
    zeros = jnp.zeros((NT, F), jnp.float32)
    b1r = b1.reshape(1, F)
    b2r = b2.reshape(1, F)

    t1, r1 = _tc_prep(x, W1[0], W1[1], root1, b1r)
    parts1 = _sc_agg_deg(t1, edata, u, zeros)
    deg = parts1[:, NP:].reshape(NC, 128 * F)[:, :N].T
    t2, r2 = _tc_mid(parts1[:, :N], deg, r1, W2[0], W2[1], root2, b2r)
    parts2 = _sc_agg(t2, edata, u, zeros)
    return _tc_fin(parts2[:, :N], deg, r2)


# speedup vs baseline: 1.0037x; 1.0037x over previous
import functools

import jax
import jax.numpy as jnp
from jax import lax
from jax.experimental import pallas as pl
from jax.experimental.pallas import tpu as pltpu
from jax.experimental.pallas import tpu_sc as plsc

N = 10000
F = 128
E = 320000
ROW = 2 * F
NC = 2
NS = 16
NW = NC * NS
EPW = E // NW
C = 80
GPC = C // 16
NCHUNK = EPW // C
NP = 10240
NT = NP + 128
RPT = NT // NS

_BLK = 1000
_GRID = N // _BLK


def _prep_body(x_ref, w0_ref, w1_ref, root_ref, b_ref, t_ref, r_ref):
    xb = x_ref[...]
    t_ref[:, :F] = jnp.dot(xb, w0_ref[...], preferred_element_type=jnp.float32)
    t_ref[:, F:] = jnp.dot(xb, w1_ref[...], preferred_element_type=jnp.float32)
    r_ref[...] = (
        jnp.dot(xb, root_ref[...], preferred_element_type=jnp.float32)
        + b_ref[...]
    )


def _tc_prep(x, w0, w1, root, b):
    return pl.pallas_call(
        _prep_body,
        grid=(_GRID,),
        in_specs=[
            pl.BlockSpec((_BLK, F), lambda i: (i, 0)),
            pl.BlockSpec((F, F), lambda i: (0, 0)),
            pl.BlockSpec((F, F), lambda i: (0, 0)),
            pl.BlockSpec((F, F), lambda i: (0, 0)),
            pl.BlockSpec((1, F), lambda i: (0, 0)),
        ],
        out_specs=[
            pl.BlockSpec((_BLK, ROW), lambda i: (i, 0)),
            pl.BlockSpec((_BLK, F), lambda i: (i, 0)),
        ],
        out_shape=[
            jax.ShapeDtypeStruct((N, ROW), jnp.float32),
            jax.ShapeDtypeStruct((N, F), jnp.float32),
        ],
    )(x, w0, w1, root, b)


def _mid_body(p_ref, d_ref, r_ref, w0_ref, w1_ref, root_ref, b_ref,
              t_ref, r2_ref):
    s = p_ref[0] + p_ref[1]
    deg = jnp.maximum(jnp.sum(d_ref[...], axis=1), 1.0)[:, None]
    h = s / deg + r_ref[...]
    t_ref[:, :F] = jnp.dot(h, w0_ref[...], preferred_element_type=jnp.float32)
    t_ref[:, F:] = jnp.dot(h, w1_ref[...], preferred_element_type=jnp.float32)
    r2_ref[...] = (
        jnp.dot(h, root_ref[...], preferred_element_type=jnp.float32)
        + b_ref[...]
    )


def _tc_mid(parts, deg, r, w0, w1, root, b):
    return pl.pallas_call(
        _mid_body,
        grid=(_GRID,),
        in_specs=[
            pl.BlockSpec((NC, _BLK, F), lambda i: (0, i, 0)),
            pl.BlockSpec((_BLK, NC), lambda i: (i, 0)),
            pl.BlockSpec((_BLK, F), lambda i: (i, 0)),
            pl.BlockSpec((F, F), lambda i: (0, 0)),
            pl.BlockSpec((F, F), lambda i: (0, 0)),
            pl.BlockSpec((F, F), lambda i: (0, 0)),
            pl.BlockSpec((1, F), lambda i: (0, 0)),
        ],
        out_specs=[
            pl.BlockSpec((_BLK, ROW), lambda i: (i, 0)),
            pl.BlockSpec((_BLK, F), lambda i: (i, 0)),
        ],
        out_shape=[
            jax.ShapeDtypeStruct((N, ROW), jnp.float32),
            jax.ShapeDtypeStruct((N, F), jnp.float32),
        ],
    )(parts, deg, r, w0, w1, root, b)


def _fin_body(p_ref, d_ref, r_ref, o_ref):
    s = p_ref[0] + p_ref[1]
    deg = jnp.maximum(jnp.sum(d_ref[...], axis=1), 1.0)[:, None]
    o_ref[...] = s / deg + r_ref[...]


def _tc_fin(parts, deg, r):
    return pl.pallas_call(
        _fin_body,
        grid=(_GRID,),
        in_specs=[
            pl.BlockSpec((NC, _BLK, F), lambda i: (0, i, 0)),
            pl.BlockSpec((_BLK, NC), lambda i: (i, 0)),
            pl.BlockSpec((_BLK, F), lambda i: (i, 0)),
        ],
        out_specs=pl.BlockSpec((_BLK, F), lambda i: (i, 0)),
        out_shape=jax.ShapeDtypeStruct((N, F), jnp.float32),
    )(parts, deg, r)


def _sc_body(with_deg, t_hbm, edata_hbm, u_hbm, zeros_hbm, out_hbm,
             eb0_v, eb1_v, ub0_v, ub1_v, gat_v, msg_v, moh_v, dch_v, didx_v,
             acc_sh, gsem, isem0, isem1):
    cid = lax.axis_index("c")
    sid = lax.axis_index("s")
    wid = sid * NC + cid
    cbase = wid * NCHUNK
    last = NCHUNK - 1

    ebufs = (eb0_v, eb1_v)
    ubufs = (ub0_v, ub1_v)
    isems = (isem0, isem1)

    def issue_idx(k, b):
        pltpu.async_copy(
            edata_hbm.at[pl.ds((cbase + k) * (2 * C), 2 * C)],
            ebufs[b], isems[b])
        pltpu.async_copy(
            u_hbm.at[pl.ds((cbase + k) * C, C)], ubufs[b], isems[b])

    def wait_idx(b):
        pltpu.make_async_copy(
            edata_hbm.at[pl.ds(cbase * (2 * C), 2 * C)],
            ebufs[b], isems[b]).wait()
        pltpu.make_async_copy(
            u_hbm.at[pl.ds(cbase * C, C)], ubufs[b], isems[b]).wait()

    def issue_gat(b):
        pltpu.async_copy(
            t_hbm.at[ebufs[b].at[pl.ds(0, C)]], gat_v, gsem)

    def wait_gat(b):
        pltpu.make_async_copy(
            t_hbm.at[ebufs[b].at[pl.ds(0, C)]], gat_v, gsem).wait()

    issue_idx(0, 0)
    issue_idx(1, 1)
    wait_idx(0)
    issue_gat(0)
    pltpu.sync_copy(zeros_hbm.at[pl.ds(sid * RPT, RPT)],
                    acc_sh.at[pl.ds(sid * RPT, RPT)])
    plsc.subcore_barrier()

    def do_chunk(c, b):
        eb_v = ebufs[b]
        ub_v = ubufs[b]
        wait_gat(b)

        def g_body(g, c2):
            dst16 = eb_v[pl.ds(C + g * 16, 16)]
            dch_v[pl.ds(g * 16, 16)] = dst16
            if with_deg:
                didx_v[pl.ds(g * 16, 16)] = jnp.right_shift(dst16, 7) + NP
                col16 = jnp.bitwise_and(dst16, 127)
            u16 = ub_v[pl.ds(g * 16, 16)]
            u16 = jnp.minimum(jnp.maximum(u16, 0.0), 1.0)
            for e2 in range(16):
                e = g * 16 + e2
                ub = jnp.full((16,), u16[e2], dtype=jnp.float32)
                for j in range(F // 16):
                    a = gat_v[e, pl.ds(j * 16, 16)]
                    bb = gat_v[e, pl.ds(F + j * 16, 16)]
                    msg_v[e, pl.ds(j * 16, 16)] = a + ub * (bb - a)
                if with_deg:
                    cb = jnp.full((16,), col16[e2], dtype=jnp.int32)
                    for j in range(F // 16):
                        lane = lax.iota(jnp.int32, 16) + (16 * j)
                        moh_v[e, pl.ds(j * 16, 16)] = jnp.where(
                            lane == cb, 1.0, 0.0)
            return c2

        lax.fori_loop(0, GPC, g_body, 0)
        wait_idx(1 - b)
        issue_gat(1 - b)
        pltpu.sync_copy(msg_v, acc_sh.at[dch_v], add=True)
        if with_deg:
            pltpu.sync_copy(moh_v, acc_sh.at[didx_v], add=True)
        issue_idx(jnp.minimum(c + 2, last), b)

    def pair_body(i, carry):
        do_chunk(2 * i, 0)
        do_chunk(2 * i + 1, 1)
        return carry

    lax.fori_loop(0, NCHUNK // 2, pair_body, 0)
    do_chunk(last, 0)
    wait_idx(0)
    wait_gat(1)
    plsc.subcore_barrier()

    pltpu.sync_copy(acc_sh.at[pl.ds(sid * RPT, RPT)],
                    out_hbm.at[cid, pl.ds(sid * RPT, RPT)])


_SC_MESH = plsc.VectorSubcoreMesh(core_axis_name="c", subcore_axis_name="s")


def _mk_scratch(with_deg):
    mrows = C if with_deg else 16
    return [
        pltpu.VMEM((2 * C,), jnp.int32),
        pltpu.VMEM((2 * C,), jnp.int32),
        pltpu.VMEM((C,), jnp.float32),
        pltpu.VMEM((C,), jnp.float32),
        pltpu.VMEM((C, ROW), jnp.float32),
        pltpu.VMEM((C, F), jnp.float32),
        pltpu.VMEM((mrows, F), jnp.float32),
        pltpu.VMEM((C,), jnp.int32),
        pltpu.VMEM((C,), jnp.int32),
        pltpu.VMEM_SHARED((NT, F), jnp.float32),
        pltpu.SemaphoreType.DMA,
        pltpu.SemaphoreType.DMA,
        pltpu.SemaphoreType.DMA,
    ]


@functools.partial(
    pl.kernel,
    mesh=_SC_MESH,
    out_type=jax.ShapeDtypeStruct((NC, NT, F), jnp.float32),
    scratch_types=_mk_scratch(True),
)
def _sc_agg_deg(t_hbm, edata_hbm, u_hbm, zeros_hbm, out_hbm, *rest):
    _sc_body(True, t_hbm, edata_hbm, u_hbm, zeros_hbm, out_hbm, *rest)


@functools.partial(
    pl.kernel,
    mesh=_SC_MESH,
    out_type=jax.ShapeDtypeStruct((NC, NT, F), jnp.float32),
    scratch_types=_mk_scratch(False),
)
def _sc_agg(t_hbm, edata_hbm, u_hbm, zeros_hbm, out_hbm, *rest):
    _sc_body(False, t_hbm, edata_hbm, u_hbm, zeros_hbm, out_hbm, *rest)


def kernel(t, x, edge_index, edge_attr, W1, root1, b1, W2, root2, b2):
    src = edge_index[0].astype(jnp.int32)
    dst = edge_index[1].astype(jnp.int32)
    u = edge_attr[:, 0].astype(jnp.float32)
    nch = E // C
    edata = jnp.stack(
        [src.reshape(nch, C), dst.reshape(nch, C)], axis=1).reshape(-1)
    zeros = jnp.zeros((NT, F), jnp.float32)
    b1r = b1.reshape(1, F)
    b2r = b2.reshape(1, F)

    t1, r1 = _tc_prep(x, W1[0], W1[1], root1, b1r)
    parts1 = _sc_agg_deg(t1, edata, u, zeros)
    deg = parts1[:, NP:].reshape(NC, 128 * F)[:, :N].T
    t2, r2 = _tc_mid(parts1[:, :N], deg, r1, W2[0], W2[1], root2, b2r)
    parts2 = _sc_agg(t2, edata, u, zeros)
    return _tc_fin(parts2[:, :N], deg, r2)
